# baseline (speedup 1.0000x reference)
import functools

import jax
import jax.numpy as jnp
from jax import lax
from jax.experimental import pallas as pl
from jax.experimental.pallas import tpu as pltpu
from jax.experimental.pallas import tpu_sc as plsc

_N = 10000
_E = 320000
_D = 128
_H = 128
_DE = 16
_INV_NORM = 1.0 / 100.0

_NC = 2
_NS = 16
_NW = _NC * _NS

_C = 128
_EH = _E // 2
_NCHH = _EH // _C
_T40 = 17
_K38 = 38
_KSTAGE = 40

_RPT = 632
_NACC = _RPT * _NS

_mesh = plsc.VectorSubcoreMesh(
    core_axis_name="c", subcore_axis_name="s", num_cores=_NC, num_subcores=_NS
)


def _half_span():
    c = lax.axis_index("c")
    s = lax.axis_index("s")
    wid = s * _NC + c
    start = jnp.where(wid < _T40, wid * _KSTAGE,
                      _T40 * _KSTAGE + (wid - _T40) * _K38)
    cnt = jnp.where(wid < _T40, _KSTAGE, _K38)
    return c, s, start, cnt


def _precompute_body(h_ref, w1_ref, a_ref, b_ref):
    hh = h_ref[...]
    w = w1_ref[...]
    a_ref[...] = jnp.dot(hh, w[0:_D, :], preferred_element_type=jnp.float32)
    b_ref[...] = jnp.dot(hh, w[_D:2 * _D, :], preferred_element_type=jnp.float32)


def _precompute(h, W1):
    bn = 2000
    return pl.pallas_call(
        _precompute_body,
        grid=(_N // bn,),
        in_specs=[
            pl.BlockSpec((bn, _D), lambda i: (i, 0)),
            pl.BlockSpec((2 * _D + _DE, _H), lambda i: (0, 0)),
        ],
        out_specs=[
            pl.BlockSpec((bn, _H), lambda i: (i, 0)),
            pl.BlockSpec((bn, _H), lambda i: (i, 0)),
        ],
        out_shape=[
            jax.ShapeDtypeStruct((_N, _H), jnp.float32),
            jax.ShapeDtypeStruct((_N, _H), jnp.float32),
        ],
    )(h, W1)


def _make_gather(offe):
    @functools.partial(
        pl.kernel,
        out_type=[
            jax.ShapeDtypeStruct((_EH, _H), jnp.float32),
            jax.ShapeDtypeStruct((_EH, _H), jnp.float32),
        ],
        mesh=_mesh,
        scratch_types=[
            pltpu.VMEM((_KSTAGE * _C,), jnp.int32),
            pltpu.VMEM((_KSTAGE * _C,), jnp.int32),
            pltpu.VMEM((_C, _H), jnp.float32),
            pltpu.VMEM((_C, _H), jnp.float32),
            pltpu.VMEM((_C, _H), jnp.float32),
            pltpu.VMEM((_C, _H), jnp.float32),
            pltpu.SemaphoreType.DMA,
            pltpu.SemaphoreType.DMA,
            pltpu.SemaphoreType.DMA,
            pltpu.SemaphoreType.DMA,
        ],
    )
    def gather(hA, hB, rows, cols, SA, SB, idxR, idxC, bA0, bB0, bA1, bB1,
               g0, g1, w0, w1):
        _, _, start, cnt = _half_span()
        pltpu.sync_copy(rows.at[pl.ds(offe + start * _C, _K38 * _C)],
                        idxR.at[pl.ds(0, _K38 * _C)])
        pltpu.sync_copy(cols.at[pl.ds(offe + start * _C, _K38 * _C)],
                        idxC.at[pl.ds(0, _K38 * _C)])

        @pl.when(cnt > _K38)
        def _():
            pltpu.sync_copy(rows.at[pl.ds(offe + (start + _K38) * _C, 2 * _C)],
                            idxR.at[pl.ds(_K38 * _C, 2 * _C)])
            pltpu.sync_copy(cols.at[pl.ds(offe + (start + _K38) * _C, 2 * _C)],
                            idxC.at[pl.ds(_K38 * _C, 2 * _C)])

        slots = ((bA0, bB0, g0, w0), (bA1, bB1, g1, w1))

        for s_ in (0, 1):
            bA, bB, g, _ = slots[s_]
            pltpu.async_copy(hA.at[idxR.at[pl.ds(s_ * _C, _C)]], bA, g)
            pltpu.async_copy(hB.at[idxC.at[pl.ds(s_ * _C, _C)]], bB, g)

        dummy = SA.at[pl.ds(0, _C)]

        def body(p, carry):
            for s_ in (0, 1):
                bA, bB, g, w = slots[s_]
                k = 2 * p + s_
                off = (start + k) * _C
                pltpu.make_async_copy(dummy, bA, g).wait()
                pltpu.make_async_copy(dummy, bB, g).wait()
                pltpu.async_copy(bA, SA.at[pl.ds(off, _C)], w)
                pltpu.async_copy(bB, SB.at[pl.ds(off, _C)], w)

                @pl.when(k + 2 < cnt)
                def _():
                    pltpu.make_async_copy(dummy, bA, w).wait()
                    pltpu.make_async_copy(dummy, bB, w).wait()
                    pltpu.async_copy(
                        hA.at[idxR.at[pl.ds((k + 2) * _C, _C)]], bA, g)
                    pltpu.async_copy(
                        hB.at[idxC.at[pl.ds((k + 2) * _C, _C)]], bB, g)

            return carry

        lax.fori_loop(0, cnt // 2, body, 0)
        for s_ in (0, 1):
            bA, bB, _, w = slots[s_]
            pltpu.make_async_copy(dummy, bA, w).wait()
            pltpu.make_async_copy(dummy, bB, w).wait()

    return gather


_gather0 = _make_gather(0)
_gather1 = _make_gather(_EH)


def _edge_body(sa_ref, sb_ref, ea_ref, w1_ref, b1_ref, w2_ref, b2_ref, o_ref):
    w1c = w1_ref[2 * _D:2 * _D + _DE, :]
    t = (
        sa_ref[...]
        + sb_ref[...]
        + jnp.dot(ea_ref[...], w1c, preferred_element_type=jnp.float32)
        + b1_ref[...]
    )
    t = t * jax.nn.sigmoid(t)
    m = (
        jnp.dot(
            t.astype(jnp.bfloat16),
            w2_ref[...].astype(jnp.bfloat16),
            preferred_element_type=jnp.float32,
        )
        + b2_ref[...]
    )
    o_ref[...] = m * jax.nn.sigmoid(m)


def _edge_mlp(SA, SB, edge_attr, W1, b1, W2, b2, offb):
    bn = 1000
    return pl.pallas_call(
        _edge_body,
        grid=(_EH // bn,),
        in_specs=[
            pl.BlockSpec((bn, _H), lambda i: (i, 0)),
            pl.BlockSpec((bn, _H), lambda i: (i, 0)),
            pl.BlockSpec((bn, _DE), lambda i: (i + offb, 0)),
            pl.BlockSpec((2 * _D + _DE, _H), lambda i: (0, 0)),
            pl.BlockSpec((1, _H), lambda i: (0, 0)),
            pl.BlockSpec((_H, _H), lambda i: (0, 0)),
            pl.BlockSpec((1, _H), lambda i: (0, 0)),
        ],
        out_specs=pl.BlockSpec((bn, _H), lambda i: (i, 0)),
        out_shape=jax.ShapeDtypeStruct((_EH, _H), jnp.float32),
    )(SA, SB, edge_attr, W1, b1, W2, b2)


def _make_scatter(offe):
    @functools.partial(
        pl.kernel,
        out_type=jax.ShapeDtypeStruct((_NC, _NACC, _H), jnp.float32),
        mesh=_mesh,
        scratch_types=[
            pltpu.VMEM((_C,), jnp.int32),
            pltpu.VMEM((_C,), jnp.int32),
            pltpu.VMEM((_C, _H), jnp.float32),
            pltpu.VMEM((_C, _H), jnp.float32),
            pltpu.VMEM_SHARED((_NACC, _H), jnp.float32),
            pltpu.SemaphoreType.DMA,
            pltpu.SemaphoreType.DMA,
        ],
    )
    def scatter(mij, rows, zeros, out, idx0, idx1, buf0, buf1, acc, m0, m1):
        c, s, start, cnt = _half_span()
        pltpu.sync_copy(zeros, acc.at[pl.ds(s * _RPT, _RPT)])
        plsc.subcore_barrier()

        slots = ((idx0, buf0, m0), (idx1, buf1, m1))
        for s_ in (0, 1):
            idx, buf, m = slots[s_]
            off = (start + s_) * _C
            pltpu.async_copy(rows.at[pl.ds(offe + off, _C)], idx, m)
            pltpu.async_copy(mij.at[pl.ds(off, _C)], buf, m)

        didx = rows.at[pl.ds(0, _C)]
        dbuf = mij.at[pl.ds(0, _C)]

        def body(p, carry):
            for s_ in (0, 1):
                idx, buf, m = slots[s_]
                k = 2 * p + s_
                pltpu.make_async_copy(didx, idx, m).wait()
                pltpu.make_async_copy(dbuf, buf, m).wait()
                pltpu.sync_copy(buf, acc.at[idx], add=True)

                @pl.when(k + 2 < cnt)
                def _():
                    off = (start + k + 2) * _C
                    pltpu.async_copy(rows.at[pl.ds(offe + off, _C)], idx, m)
                    pltpu.async_copy(mij.at[pl.ds(off, _C)], buf, m)

            return carry

        lax.fori_loop(0, cnt // 2, body, 0)
        plsc.subcore_barrier()
        pltpu.sync_copy(
            acc.at[pl.ds(s * _RPT, _RPT)], out.at[c, pl.ds(s * _RPT, _RPT)]
        )

    return scatter


_scatter0 = _make_scatter(0)
_scatter1 = _make_scatter(_EH)


def _node_body(h_ref, p0_ref, p1_ref, w3_ref, b3_ref, w4_ref, b4_ref, o_ref):
    p0 = p0_ref[...]
    p1 = p1_ref[...]
    agg = (p0[0] + p0[1] + p1[0] + p1[1]) * _INV_NORM
    hh = h_ref[...]
    x = (
        jnp.dot(hh, w3_ref[0:_D, :], preferred_element_type=jnp.float32)
        + jnp.dot(agg, w3_ref[_D:_D + _H, :], preferred_element_type=jnp.float32)
        + b3_ref[...]
    )
    u = x * jax.nn.sigmoid(x)
    o_ref[...] = (
        hh + jnp.dot(u, w4_ref[...], preferred_element_type=jnp.float32) + b4_ref[...]
    )


def _node_mlp(h, p0, p1, W3, b3, W4, b4):
    bn = 1000
    return pl.pallas_call(
        _node_body,
        grid=(_N // bn,),
        in_specs=[
            pl.BlockSpec((bn, _D), lambda i: (i, 0)),
            pl.BlockSpec((_NC, bn, _H), lambda i: (0, i, 0)),
            pl.BlockSpec((_NC, bn, _H), lambda i: (0, i, 0)),
            pl.BlockSpec((_H + _D, _H), lambda i: (0, 0)),
            pl.BlockSpec((1, _H), lambda i: (0, 0)),
            pl.BlockSpec((_H, _D), lambda i: (0, 0)),
            pl.BlockSpec((1, _D), lambda i: (0, 0)),
        ],
        out_specs=pl.BlockSpec((bn, _D), lambda i: (i, 0)),
        out_shape=jax.ShapeDtypeStruct((_N, _D), jnp.float32),
    )(h, p0, p1, W3, b3, W4, b4)


def kernel(h, edge_index, edge_attr, W1, b1, W2, b2, W3, b3, W4, b4):
    row = edge_index[0].astype(jnp.int32)
    col = edge_index[1].astype(jnp.int32)

    hA, hB = _precompute(h, W1)
    SA0, SB0 = _gather0(hA, hB, row, col)
    SA1, SB1 = _gather1(hA, hB, row, col)
    b1r = b1.reshape(1, _H)
    b2r = b2.reshape(1, _H)
    m0 = _edge_mlp(SA0, SB0, edge_attr, W1, b1r, W2, b2r, 0)
    m1 = _edge_mlp(SA1, SB1, edge_attr, W1, b1r, W2, b2r, _EH // 1000)
    zeros = jnp.zeros((_RPT, _H), jnp.float32)
    p0 = _scatter0(m0, row, zeros)
    p1 = _scatter1(m1, row, zeros)
    mij = jnp.concatenate([m0, m1], axis=0)
    h_out = _node_mlp(h, p0, p1, W3, b3.reshape(1, _H), W4, b4.reshape(1, _D))
    return h_out, mij


# speedup vs baseline: 1.1022x; 1.1022x over previous
import functools

import jax
import jax.numpy as jnp
from jax import lax
from jax.experimental import pallas as pl
from jax.experimental.pallas import tpu as pltpu
from jax.experimental.pallas import tpu_sc as plsc

_N = 10000
_E = 320000
_D = 128
_H = 128
_DE = 16
_INV_NORM = 1.0 / 100.0

_NC = 2
_NS = 16
_NW = _NC * _NS

_C = 128
_EH = _E // 2
_NCHH = _EH // _C
_T40 = 17
_K38 = 38
_KSTAGE = 40

_RPT = 632
_NACC = _RPT * _NS

_mesh = plsc.VectorSubcoreMesh(
    core_axis_name="c", subcore_axis_name="s", num_cores=_NC, num_subcores=_NS
)


def _half_span():
    c = lax.axis_index("c")
    s = lax.axis_index("s")
    wid = s * _NC + c
    start = jnp.where(wid < _T40, wid * _KSTAGE,
                      _T40 * _KSTAGE + (wid - _T40) * _K38)
    cnt = jnp.where(wid < _T40, _KSTAGE, _K38)
    return c, s, start, cnt


def _precompute_body(h_ref, w1_ref, a_ref, b_ref):
    hh = h_ref[...]
    w = w1_ref[...]
    a_ref[...] = jnp.dot(hh, w[0:_D, :], preferred_element_type=jnp.float32)
    b_ref[...] = jnp.dot(hh, w[_D:2 * _D, :], preferred_element_type=jnp.float32)


def _precompute(h, W1):
    bn = 2000
    return pl.pallas_call(
        _precompute_body,
        grid=(_N // bn,),
        in_specs=[
            pl.BlockSpec((bn, _D), lambda i: (i, 0)),
            pl.BlockSpec((2 * _D + _DE, _H), lambda i: (0, 0)),
        ],
        out_specs=[
            pl.BlockSpec((bn, _H), lambda i: (i, 0)),
            pl.BlockSpec((bn, _H), lambda i: (i, 0)),
        ],
        out_shape=[
            jax.ShapeDtypeStruct((_N, _H), jnp.float32),
            jax.ShapeDtypeStruct((_N, _H), jnp.float32),
        ],
    )(h, W1)


def _make_gather(offe):
    @functools.partial(
        pl.kernel,
        out_type=[
            jax.ShapeDtypeStruct((_EH, _H), jnp.float32),
            jax.ShapeDtypeStruct((_EH, _H), jnp.float32),
        ],
        mesh=_mesh,
        scratch_types=[
            pltpu.VMEM((_KSTAGE * _C,), jnp.int32),
            pltpu.VMEM((_KSTAGE * _C,), jnp.int32),
            pltpu.VMEM((_C, _H), jnp.float32),
            pltpu.VMEM((_C, _H), jnp.float32),
            pltpu.VMEM((_C, _H), jnp.float32),
            pltpu.VMEM((_C, _H), jnp.float32),
            pltpu.SemaphoreType.DMA,
            pltpu.SemaphoreType.DMA,
            pltpu.SemaphoreType.DMA,
            pltpu.SemaphoreType.DMA,
        ],
    )
    def gather(hA, hB, rows, cols, SA, SB, idxR, idxC, bA0, bB0, bA1, bB1,
               g0, g1, w0, w1):
        _, _, start, cnt = _half_span()
        pltpu.sync_copy(rows.at[pl.ds(offe + start * _C, _K38 * _C)],
                        idxR.at[pl.ds(0, _K38 * _C)])
        pltpu.sync_copy(cols.at[pl.ds(offe + start * _C, _K38 * _C)],
                        idxC.at[pl.ds(0, _K38 * _C)])

        @pl.when(cnt > _K38)
        def _():
            pltpu.sync_copy(rows.at[pl.ds(offe + (start + _K38) * _C, 2 * _C)],
                            idxR.at[pl.ds(_K38 * _C, 2 * _C)])
            pltpu.sync_copy(cols.at[pl.ds(offe + (start + _K38) * _C, 2 * _C)],
                            idxC.at[pl.ds(_K38 * _C, 2 * _C)])

        slots = ((bA0, bB0, g0, w0), (bA1, bB1, g1, w1))

        for s_ in (0, 1):
            bA, bB, g, _ = slots[s_]
            pltpu.async_copy(hA.at[idxR.at[pl.ds(s_ * _C, _C)]], bA, g)
            pltpu.async_copy(hB.at[idxC.at[pl.ds(s_ * _C, _C)]], bB, g)

        dummy = SA.at[pl.ds(0, _C)]

        def body(p, carry):
            for s_ in (0, 1):
                bA, bB, g, w = slots[s_]
                k = 2 * p + s_
                off = (start + k) * _C
                pltpu.make_async_copy(dummy, bA, g).wait()
                pltpu.make_async_copy(dummy, bB, g).wait()
                pltpu.async_copy(bA, SA.at[pl.ds(off, _C)], w)
                pltpu.async_copy(bB, SB.at[pl.ds(off, _C)], w)

                @pl.when(k + 2 < cnt)
                def _():
                    pltpu.make_async_copy(dummy, bA, w).wait()
                    pltpu.make_async_copy(dummy, bB, w).wait()
                    pltpu.async_copy(
                        hA.at[idxR.at[pl.ds((k + 2) * _C, _C)]], bA, g)
                    pltpu.async_copy(
                        hB.at[idxC.at[pl.ds((k + 2) * _C, _C)]], bB, g)

            return carry

        lax.fori_loop(0, cnt // 2, body, 0)
        for s_ in (0, 1):
            bA, bB, _, w = slots[s_]
            pltpu.make_async_copy(dummy, bA, w).wait()
            pltpu.make_async_copy(dummy, bB, w).wait()

    return gather


_gather0 = _make_gather(0)
_gather1 = _make_gather(_EH)


def _edge_body(sa_ref, sb_ref, ea_ref, w1_ref, b1_ref, w2_ref, b2_ref, o_ref):
    w1c = w1_ref[2 * _D:2 * _D + _DE, :]
    t = (
        sa_ref[...]
        + sb_ref[...]
        + jnp.dot(ea_ref[...], w1c, preferred_element_type=jnp.float32)
        + b1_ref[...]
    ).astype(jnp.bfloat16)
    t = t * jax.nn.sigmoid(t)
    m = (
        jnp.dot(
            t,
            w2_ref[...].astype(jnp.bfloat16),
            preferred_element_type=jnp.float32,
        )
        + b2_ref[...]
    )
    o_ref[...] = m * jax.nn.sigmoid(m)


def _edge_mlp(SA, SB, edge_attr, W1, b1, W2, b2, half):
    bn = 2000
    offb = half * (_EH // bn)
    return pl.pallas_call(
        _edge_body,
        grid=(_EH // bn,),
        in_specs=[
            pl.BlockSpec((bn, _H), lambda i: (i, 0)),
            pl.BlockSpec((bn, _H), lambda i: (i, 0)),
            pl.BlockSpec((bn, _DE), lambda i: (i + offb, 0)),
            pl.BlockSpec((2 * _D + _DE, _H), lambda i: (0, 0)),
            pl.BlockSpec((1, _H), lambda i: (0, 0)),
            pl.BlockSpec((_H, _H), lambda i: (0, 0)),
            pl.BlockSpec((1, _H), lambda i: (0, 0)),
        ],
        out_specs=pl.BlockSpec((bn, _H), lambda i: (i, 0)),
        out_shape=jax.ShapeDtypeStruct((_EH, _H), jnp.float32),
    )(SA, SB, edge_attr, W1, b1, W2, b2)


def _make_scatter(offe):
    @functools.partial(
        pl.kernel,
        out_type=jax.ShapeDtypeStruct((_NC, _NACC, _H), jnp.float32),
        mesh=_mesh,
        scratch_types=[
            pltpu.VMEM((_C,), jnp.int32),
            pltpu.VMEM((_C,), jnp.int32),
            pltpu.VMEM((_C, _H), jnp.float32),
            pltpu.VMEM((_C, _H), jnp.float32),
            pltpu.VMEM_SHARED((_NACC, _H), jnp.float32),
            pltpu.SemaphoreType.DMA,
            pltpu.SemaphoreType.DMA,
        ],
    )
    def scatter(mij, rows, zeros, out, idx0, idx1, buf0, buf1, acc, m0, m1):
        c, s, start, cnt = _half_span()
        pltpu.sync_copy(zeros, acc.at[pl.ds(s * _RPT, _RPT)])
        plsc.subcore_barrier()

        slots = ((idx0, buf0, m0), (idx1, buf1, m1))
        for s_ in (0, 1):
            idx, buf, m = slots[s_]
            off = (start + s_) * _C
            pltpu.async_copy(rows.at[pl.ds(offe + off, _C)], idx, m)
            pltpu.async_copy(mij.at[pl.ds(off, _C)], buf, m)

        didx = rows.at[pl.ds(0, _C)]
        dbuf = mij.at[pl.ds(0, _C)]

        def body(p, carry):
            for s_ in (0, 1):
                idx, buf, m = slots[s_]
                k = 2 * p + s_
                pltpu.make_async_copy(didx, idx, m).wait()
                pltpu.make_async_copy(dbuf, buf, m).wait()
                pltpu.sync_copy(buf, acc.at[idx], add=True)

                @pl.when(k + 2 < cnt)
                def _():
                    off = (start + k + 2) * _C
                    pltpu.async_copy(rows.at[pl.ds(offe + off, _C)], idx, m)
                    pltpu.async_copy(mij.at[pl.ds(off, _C)], buf, m)

            return carry

        lax.fori_loop(0, cnt // 2, body, 0)
        plsc.subcore_barrier()
        pltpu.sync_copy(
            acc.at[pl.ds(s * _RPT, _RPT)], out.at[c, pl.ds(s * _RPT, _RPT)]
        )

    return scatter


_scatter0 = _make_scatter(0)
_scatter1 = _make_scatter(_EH)


def _node_body(h_ref, p0_ref, p1_ref, w3_ref, b3_ref, w4_ref, b4_ref, o_ref):
    p0 = p0_ref[...]
    p1 = p1_ref[...]
    agg = (p0[0] + p0[1] + p1[0] + p1[1]) * _INV_NORM
    hh = h_ref[...]
    x = (
        jnp.dot(hh, w3_ref[0:_D, :], preferred_element_type=jnp.float32)
        + jnp.dot(agg, w3_ref[_D:_D + _H, :], preferred_element_type=jnp.float32)
        + b3_ref[...]
    )
    u = x * jax.nn.sigmoid(x)
    o_ref[...] = (
        hh + jnp.dot(u, w4_ref[...], preferred_element_type=jnp.float32) + b4_ref[...]
    )


def _node_mlp(h, p0, p1, W3, b3, W4, b4):
    bn = 1000
    return pl.pallas_call(
        _node_body,
        grid=(_N // bn,),
        in_specs=[
            pl.BlockSpec((bn, _D), lambda i: (i, 0)),
            pl.BlockSpec((_NC, bn, _H), lambda i: (0, i, 0)),
            pl.BlockSpec((_NC, bn, _H), lambda i: (0, i, 0)),
            pl.BlockSpec((_H + _D, _H), lambda i: (0, 0)),
            pl.BlockSpec((1, _H), lambda i: (0, 0)),
            pl.BlockSpec((_H, _D), lambda i: (0, 0)),
            pl.BlockSpec((1, _D), lambda i: (0, 0)),
        ],
        out_specs=pl.BlockSpec((bn, _D), lambda i: (i, 0)),
        out_shape=jax.ShapeDtypeStruct((_N, _D), jnp.float32),
    )(h, p0, p1, W3, b3, W4, b4)


def kernel(h, edge_index, edge_attr, W1, b1, W2, b2, W3, b3, W4, b4):
    row = edge_index[0].astype(jnp.int32)
    col = edge_index[1].astype(jnp.int32)

    hA, hB = _precompute(h, W1)
    SA0, SB0 = _gather0(hA, hB, row, col)
    SA1, SB1 = _gather1(hA, hB, row, col)
    b1r = b1.reshape(1, _H)
    b2r = b2.reshape(1, _H)
    m0 = _edge_mlp(SA0, SB0, edge_attr, W1, b1r, W2, b2r, 0)
    m1 = _edge_mlp(SA1, SB1, edge_attr, W1, b1r, W2, b2r, 1)
    zeros = jnp.zeros((_RPT, _H), jnp.float32)
    p0 = _scatter0(m0, row, zeros)
    p1 = _scatter1(m1, row, zeros)
    mij = jnp.concatenate([m0, m1], axis=0)
    h_out = _node_mlp(h, p0, p1, W3, b3.reshape(1, _H), W4, b4.reshape(1, _D))
    return h_out, mij
